# baseline (speedup 1.0000x reference)
import functools

import jax
import jax.numpy as jnp
from jax import lax
from jax.experimental import pallas as pl
from jax.experimental.pallas import tpu as pltpu
from jax.experimental.pallas import tpu_sc as plsc

F = 14.399645

NA = 100000
NP = 6400000
NM = 1000

NC, NS, L = 2, 16, 16
NW = NC * NS

NA_PAD = 131072
CHUNK = 1024
ROWS_PER_CHUNK = CHUNK // 128
NCH_TOTAL = NP // CHUNK
NCH_MAIN = 195
NCH_PIPE = 192
PPW = NCH_MAIN * CHUNK
NCH_EXTRA = NCH_TOTAL - NW * NCH_MAIN

NM_PAD = 1024
APW = NA_PAD // NW
BROWS = APW // 128

_mesh = plsc.VectorSubcoreMesh(core_axis_name="c", subcore_axis_name="s")


@functools.partial(
    pl.kernel,
    mesh=_mesh,
    out_type=(
        jax.ShapeDtypeStruct((NC * NA_PAD,), jnp.float32),
        jax.ShapeDtypeStruct((NC * NM_PAD,), jnp.float32),
    ),
    compiler_params=pltpu.CompilerParams(needs_layout_passes=False),
    scratch_types=[
        pltpu.VMEM((NA,), jnp.float32),
        pltpu.VMEM((CHUNK,), jnp.int32),
        pltpu.VMEM((CHUNK,), jnp.int32),
        pltpu.VMEM((CHUNK,), jnp.float32),
        pltpu.VMEM((CHUNK,), jnp.float32),
        pltpu.VMEM((ROWS_PER_CHUNK, 128), jnp.int32),
        pltpu.VMEM((ROWS_PER_CHUNK, 128), jnp.int32),
        pltpu.VMEM((ROWS_PER_CHUNK, 128), jnp.int32),
        pltpu.VMEM((ROWS_PER_CHUNK, 128), jnp.int32),
        pltpu.VMEM((CHUNK,), jnp.float32),
        pltpu.VMEM((CHUNK,), jnp.float32),
        pltpu.VMEM((CHUNK,), jnp.float32),
        pltpu.VMEM((CHUNK,), jnp.float32),
        pltpu.VMEM_SHARED((NA_PAD,), jnp.float32),
        pltpu.VMEM_SHARED((NM_PAD,), jnp.float32),
        pltpu.SemaphoreType.DMA,
        pltpu.SemaphoreType.DMA,
        pltpu.SemaphoreType.DMA,
        pltpu.SemaphoreType.DMA,
        pltpu.SemaphoreType.DMA,
        pltpu.SemaphoreType.DMA,
    ],
)
def _pairs_kernel(q_hbm, dist_hbm, first_hbm, sec_hbm, mol_hbm, zeros_hbm,
                  part_hbm, bins_hbm,
                  ch_v, sec0_v, sec1_v, dist0_v, dist1_v,
                  first0_v, first1_v, first2_v, first3_v,
                  v0_v, v1_v, v2_v, v3_v, acc_sh, bins_sh,
                  isem0, isem1, ssem0, ssem1, ssem2, ssem3):
    cid = lax.axis_index("c")
    sid = lax.axis_index("s")
    wid = cid * NS + sid
    sec_v = (sec0_v, sec1_v)
    dist_v = (dist0_v, dist1_v)
    first_v = (first0_v, first1_v, first2_v, first3_v)
    v_v = (v0_v, v1_v, v2_v, v3_v)
    isems = (isem0, isem1)
    ssems = (ssem0, ssem1, ssem2, ssem3)

    pltpu.sync_copy(q_hbm.at[pl.ds(0, NA)], ch_v)

    @pl.when(sid == 0)
    def _():
        pltpu.sync_copy(zeros_hbm, acc_sh)
        pltpu.sync_copy(zeros_hbm.at[pl.ds(0, NM_PAD)], bins_sh)

    plsc.subcore_barrier()

    def chunk_off(k):
        return wid * PPW + k * CHUNK

    def chunk_row(k):
        return wid * (PPW // 128) + k * ROWS_PER_CHUNK

    def extra_off():
        return (NW * NCH_MAIN + wid) * CHUNK

    def extra_row():
        return NW * NCH_MAIN * ROWS_PER_CHUNK + wid * ROWS_PER_CHUNK

    def start_inputs(off, row, b2, b4):
        pltpu.async_copy(sec_hbm.at[pl.ds(off, CHUNK)], sec_v[b2], isems[b2])
        pltpu.async_copy(dist_hbm.at[pl.ds(off, CHUNK)], dist_v[b2], isems[b2])
        pltpu.async_copy(first_hbm.at[pl.ds(row, ROWS_PER_CHUNK), :],
                         first_v[b4], isems[b2])

    def wait_inputs(off, row, b2, b4):
        pltpu.make_async_copy(sec_hbm.at[pl.ds(off, CHUNK)], sec_v[b2],
                              isems[b2]).wait()
        pltpu.make_async_copy(dist_hbm.at[pl.ds(off, CHUNK)], dist_v[b2],
                              isems[b2]).wait()
        pltpu.make_async_copy(first_hbm.at[pl.ds(row, ROWS_PER_CHUNK), :],
                              first_v[b4], isems[b2]).wait()

    def drain_scatter(b4):
        for j in range(ROWS_PER_CHUNK):
            pltpu.make_async_copy(v_v[b4].at[pl.ds(j * 128, 128)],
                                  acc_sh.at[first_v[b4].at[j]],
                                  ssems[b4]).wait()

    def compute(b2, b4):
        sv, dv, vv = sec_v[b2], dist_v[b2], v_v[b4]

        def inner(i, c):
            s = pl.ds(i * L, L)
            idx = sv[s]
            q = plsc.load_gather(ch_v, [idx])
            d = dv[s]
            vv[s] = (F * q) / d
            return c

        lax.fori_loop(0, CHUNK // L, inner, 0, unroll=4)

    def issue_scatter(b4):
        for j in range(ROWS_PER_CHUNK):
            pltpu.async_copy(v_v[b4].at[pl.ds(j * 128, 128)],
                             acc_sh.at[first_v[b4].at[j]], ssems[b4],
                             add=True)

    start_inputs(chunk_off(0), chunk_row(0), 0, 0)

    def pair_body(t, carry):
        for u in range(4):
            k = 4 * t + u
            b2, b4 = u % 2, u
            wait_inputs(chunk_off(k), chunk_row(k), b2, b4)

            @pl.when(k >= 3)
            def _():
                drain_scatter((u + 1) % 4)

            @pl.when(k + 1 < NCH_PIPE)
            def _():
                start_inputs(chunk_off(k + 1), chunk_row(k + 1),
                             (u + 1) % 2, (u + 1) % 4)

            compute(b2, b4)
            issue_scatter(b4)
        return carry

    lax.fori_loop(0, NCH_PIPE // 4, pair_body, 0)
    drain_scatter(1)
    drain_scatter(2)
    drain_scatter(3)

    for j, k in enumerate(range(NCH_PIPE, NCH_MAIN)):
        pltpu.sync_copy(sec_hbm.at[pl.ds(chunk_off(k), CHUNK)], sec_v[j % 2])
        pltpu.sync_copy(dist_hbm.at[pl.ds(chunk_off(k), CHUNK)], dist_v[j % 2])
        pltpu.sync_copy(first_hbm.at[pl.ds(chunk_row(k), ROWS_PER_CHUNK), :],
                        first_v[j])
        compute(j % 2, j)
        issue_scatter(j)

    @pl.when(wid < NCH_EXTRA)
    def _():
        pltpu.sync_copy(sec_hbm.at[pl.ds(extra_off(), CHUNK)], sec_v[0])
        pltpu.sync_copy(dist_hbm.at[pl.ds(extra_off(), CHUNK)], dist_v[0])
        pltpu.sync_copy(first_hbm.at[pl.ds(extra_row(), ROWS_PER_CHUNK), :],
                        first_v[3])
        compute(0, 3)
        issue_scatter(3)

    drain_scatter(0)
    drain_scatter(1)
    drain_scatter(2)

    @pl.when(wid < NCH_EXTRA)
    def _():
        drain_scatter(3)

    plsc.subcore_barrier()

    seg = NA_PAD // NS
    acc_dump = pltpu.async_copy(
        acc_sh.at[pl.ds(sid * seg, seg)],
        part_hbm.at[pl.ds(cid * NA_PAD + sid * seg, seg)], isems[0])

    def drain_bins(b):
        for j in range(ROWS_PER_CHUNK):
            pltpu.make_async_copy(v_v[2 + b].at[pl.ds(j * 128, 128)],
                                  bins_sh.at[first_v[b].at[j]],
                                  ssems[b]).wait()

    for m in range(seg // CHUNK):
        b = m % 2
        abase = sid * seg + m * CHUNK
        arow = sid * (seg // 128) + m * ROWS_PER_CHUNK
        if m >= 2:
            drain_bins(b)
        pltpu.sync_copy(acc_sh.at[pl.ds(abase, CHUNK)], v_v[b])
        pltpu.sync_copy(q_hbm.at[pl.ds(abase, CHUNK)], dist_v[b])
        pltpu.sync_copy(mol_hbm.at[pl.ds(arow, ROWS_PER_CHUNK), :], first_v[b])
        va, vq, vc = v_v[b], dist_v[b], v_v[2 + b]

        def binner(i, c):
            s = pl.ds(i * L, L)
            vc[s] = (0.5 * va[s]) * vq[s]
            return c

        lax.fori_loop(0, CHUNK // L, binner, 0, unroll=4)
        for j in range(ROWS_PER_CHUNK):
            pltpu.async_copy(v_v[2 + b].at[pl.ds(j * 128, 128)],
                             bins_sh.at[first_v[b].at[j]], ssems[b], add=True)

    drain_bins(0)
    drain_bins(1)
    acc_dump.wait()
    plsc.subcore_barrier()

    @pl.when(sid == 0)
    def _():
        pltpu.sync_copy(bins_sh, bins_hbm.at[pl.ds(cid * NM_PAD, NM_PAD)])


@functools.partial(
    pl.kernel,
    mesh=_mesh,
    out_type=jax.ShapeDtypeStruct((NA_PAD,), jnp.float32),
    compiler_params=pltpu.CompilerParams(needs_layout_passes=False),
    scratch_types=[
        pltpu.VMEM((APW,), jnp.float32),
        pltpu.VMEM((APW,), jnp.float32),
        pltpu.VMEM((APW,), jnp.float32),
    ],
)
def _volt_kernel(part_hbm, volt_hbm, p0_v, p1_v, v_v):
    cid = lax.axis_index("c")
    sid = lax.axis_index("s")
    wid = cid * NS + sid
    base = wid * APW

    pltpu.sync_copy(part_hbm.at[pl.ds(base, APW)], p0_v)
    pltpu.sync_copy(part_hbm.at[pl.ds(NA_PAD + base, APW)], p1_v)

    def inner(i, c):
        s = pl.ds(i * L, L)
        v_v[s] = p0_v[s] + p1_v[s]
        return c

    lax.fori_loop(0, APW // L, inner, 0, unroll=4)

    pltpu.sync_copy(v_v, volt_hbm.at[pl.ds(base, APW)])


def kernel(charges, pair_dist, pair_first, pair_second, mol_index, n_molecules):
    q = charges.reshape(NA)
    first_2d = pair_first.reshape(NP // 128, 128)
    pada = NA_PAD - NA
    q_pad = jnp.concatenate([q, jnp.zeros((pada,), jnp.float32)])
    mol_p = jnp.concatenate(
        [mol_index, jnp.full((pada,), NM_PAD - 1, jnp.int32)]).reshape(NA_PAD // 128, 128)
    zeros_acc = jnp.zeros((NA_PAD,), jnp.float32)

    part, bins = _pairs_kernel(q_pad, pair_dist, first_2d, pair_second,
                               mol_p, zeros_acc)
    volt = _volt_kernel(part)

    voltage_atom = volt[:NA].reshape(NA, 1)
    coulomb_molecules = (bins[:NM] + bins[NM_PAD:NM_PAD + NM]).reshape(NM, 1)
    return (coulomb_molecules, voltage_atom)


# speedup vs baseline: 1.0023x; 1.0023x over previous
import functools

import jax
import jax.numpy as jnp
from jax import lax
from jax.experimental import pallas as pl
from jax.experimental.pallas import tpu as pltpu
from jax.experimental.pallas import tpu_sc as plsc

F = 14.399645

NA = 100000
NP = 6400000
NM = 1000

NC, NS, L = 2, 16, 16
NW = NC * NS

NA_PAD = 131072
CHUNK = 1024
ROWS_PER_CHUNK = CHUNK // 128
NCH_TOTAL = NP // CHUNK
NCH_MAIN = 195
NCH_PIPE = 192
PPW = NCH_MAIN * CHUNK
NCH_EXTRA = NCH_TOTAL - NW * NCH_MAIN

NM_PAD = 1024
APW = NA_PAD // NW
BROWS = APW // 128

_mesh = plsc.VectorSubcoreMesh(core_axis_name="c", subcore_axis_name="s")


@functools.partial(
    pl.kernel,
    mesh=_mesh,
    out_type=(
        jax.ShapeDtypeStruct((NC * NA_PAD,), jnp.float32),
        jax.ShapeDtypeStruct((NC * NM_PAD,), jnp.float32),
    ),
    compiler_params=pltpu.CompilerParams(needs_layout_passes=False),
    scratch_types=[
        pltpu.VMEM((NA,), jnp.float32),
        pltpu.VMEM((CHUNK,), jnp.int32),
        pltpu.VMEM((CHUNK,), jnp.int32),
        pltpu.VMEM((CHUNK,), jnp.float32),
        pltpu.VMEM((CHUNK,), jnp.float32),
        pltpu.VMEM((ROWS_PER_CHUNK, 128), jnp.int32),
        pltpu.VMEM((ROWS_PER_CHUNK, 128), jnp.int32),
        pltpu.VMEM((ROWS_PER_CHUNK, 128), jnp.int32),
        pltpu.VMEM((ROWS_PER_CHUNK, 128), jnp.int32),
        pltpu.VMEM((CHUNK,), jnp.float32),
        pltpu.VMEM((CHUNK,), jnp.float32),
        pltpu.VMEM((CHUNK,), jnp.float32),
        pltpu.VMEM((CHUNK,), jnp.float32),
        pltpu.VMEM_SHARED((NA_PAD,), jnp.float32),
        pltpu.VMEM_SHARED((NM_PAD,), jnp.float32),
        pltpu.SemaphoreType.DMA,
        pltpu.SemaphoreType.DMA,
        pltpu.SemaphoreType.DMA,
        pltpu.SemaphoreType.DMA,
        pltpu.SemaphoreType.DMA,
        pltpu.SemaphoreType.DMA,
        pltpu.SemaphoreType.DMA,
    ],
)
def _pairs_kernel(q_hbm, dist_hbm, first_hbm, sec_hbm, mol_hbm, zeros_hbm,
                  part_hbm, bins_hbm,
                  ch_v, sec0_v, sec1_v, dist0_v, dist1_v,
                  first0_v, first1_v, first2_v, first3_v,
                  v0_v, v1_v, v2_v, v3_v, acc_sh, bins_sh,
                  isem0, isem1, ssem0, ssem1, ssem2, ssem3, dsem):
    cid = lax.axis_index("c")
    sid = lax.axis_index("s")
    wid = cid * NS + sid
    sec_v = (sec0_v, sec1_v)
    dist_v = (dist0_v, dist1_v)
    first_v = (first0_v, first1_v, first2_v, first3_v)
    v_v = (v0_v, v1_v, v2_v, v3_v)
    isems = (isem0, isem1)
    ssems = (ssem0, ssem1, ssem2, ssem3)

    pltpu.sync_copy(q_hbm.at[pl.ds(0, NA)], ch_v)

    @pl.when(sid == 0)
    def _():
        pltpu.sync_copy(zeros_hbm, acc_sh)
        pltpu.sync_copy(zeros_hbm.at[pl.ds(0, NM_PAD)], bins_sh)

    plsc.subcore_barrier()

    def chunk_off(k):
        return wid * PPW + k * CHUNK

    def chunk_row(k):
        return wid * (PPW // 128) + k * ROWS_PER_CHUNK

    def extra_off():
        return (NW * NCH_MAIN + wid) * CHUNK

    def extra_row():
        return NW * NCH_MAIN * ROWS_PER_CHUNK + wid * ROWS_PER_CHUNK

    def start_inputs(off, row, b2, b4):
        pltpu.async_copy(sec_hbm.at[pl.ds(off, CHUNK)], sec_v[b2], isems[b2])
        pltpu.async_copy(dist_hbm.at[pl.ds(off, CHUNK)], dist_v[b2], isems[b2])
        pltpu.async_copy(first_hbm.at[pl.ds(row, ROWS_PER_CHUNK), :],
                         first_v[b4], isems[b2])

    def wait_inputs(off, row, b2, b4):
        pltpu.make_async_copy(sec_hbm.at[pl.ds(off, CHUNK)], sec_v[b2],
                              isems[b2]).wait()
        pltpu.make_async_copy(dist_hbm.at[pl.ds(off, CHUNK)], dist_v[b2],
                              isems[b2]).wait()
        pltpu.make_async_copy(first_hbm.at[pl.ds(row, ROWS_PER_CHUNK), :],
                              first_v[b4], isems[b2]).wait()

    def drain_scatter(b4):
        for j in range(ROWS_PER_CHUNK):
            pltpu.make_async_copy(v_v[b4].at[pl.ds(j * 128, 128)],
                                  acc_sh.at[first_v[b4].at[j]],
                                  ssems[b4]).wait()

    def compute(b2, b4):
        sv, dv, vv = sec_v[b2], dist_v[b2], v_v[b4]

        def inner(i, c):
            s = pl.ds(i * L, L)
            idx = sv[s]
            q = plsc.load_gather(ch_v, [idx])
            d = dv[s]
            vv[s] = (F * q) / d
            return c

        lax.fori_loop(0, CHUNK // L, inner, 0, unroll=4)

    def issue_scatter(b4):
        for j in range(ROWS_PER_CHUNK):
            pltpu.async_copy(v_v[b4].at[pl.ds(j * 128, 128)],
                             acc_sh.at[first_v[b4].at[j]], ssems[b4],
                             add=True)

    start_inputs(chunk_off(0), chunk_row(0), 0, 0)

    def pair_body(t, carry):
        for u in range(4):
            k = 4 * t + u
            b2, b4 = u % 2, u
            wait_inputs(chunk_off(k), chunk_row(k), b2, b4)

            @pl.when(k >= 3)
            def _():
                drain_scatter((u + 1) % 4)

            @pl.when(k + 1 < NCH_PIPE)
            def _():
                start_inputs(chunk_off(k + 1), chunk_row(k + 1),
                             (u + 1) % 2, (u + 1) % 4)

            compute(b2, b4)
            issue_scatter(b4)
        return carry

    lax.fori_loop(0, NCH_PIPE // 4, pair_body, 0)
    drain_scatter(1)
    drain_scatter(2)
    drain_scatter(3)

    for j, k in enumerate(range(NCH_PIPE, NCH_MAIN)):
        pltpu.sync_copy(sec_hbm.at[pl.ds(chunk_off(k), CHUNK)], sec_v[j % 2])
        pltpu.sync_copy(dist_hbm.at[pl.ds(chunk_off(k), CHUNK)], dist_v[j % 2])
        pltpu.sync_copy(first_hbm.at[pl.ds(chunk_row(k), ROWS_PER_CHUNK), :],
                        first_v[j])
        compute(j % 2, j)
        issue_scatter(j)

    @pl.when(wid < NCH_EXTRA)
    def _():
        pltpu.sync_copy(sec_hbm.at[pl.ds(extra_off(), CHUNK)], sec_v[0])
        pltpu.sync_copy(dist_hbm.at[pl.ds(extra_off(), CHUNK)], dist_v[0])
        pltpu.sync_copy(first_hbm.at[pl.ds(extra_row(), ROWS_PER_CHUNK), :],
                        first_v[3])
        compute(0, 3)
        issue_scatter(3)

    drain_scatter(0)
    drain_scatter(1)
    drain_scatter(2)

    @pl.when(wid < NCH_EXTRA)
    def _():
        drain_scatter(3)

    plsc.subcore_barrier()

    seg = NA_PAD // NS
    acc_dump = pltpu.async_copy(
        acc_sh.at[pl.ds(sid * seg, seg)],
        part_hbm.at[pl.ds(cid * NA_PAD + sid * seg, seg)], dsem)

    MSUB = seg // CHUNK

    def bin_start(m, b2, b4):
        abase = sid * seg + m * CHUNK
        arow = sid * (seg // 128) + m * ROWS_PER_CHUNK
        pltpu.async_copy(q_hbm.at[pl.ds(abase, CHUNK)], dist_v[b2], isems[b2])
        pltpu.async_copy(mol_hbm.at[pl.ds(arow, ROWS_PER_CHUNK), :],
                         first_v[b4], isems[b2])

    def bin_wait(m, b2, b4):
        abase = sid * seg + m * CHUNK
        arow = sid * (seg // 128) + m * ROWS_PER_CHUNK
        pltpu.make_async_copy(q_hbm.at[pl.ds(abase, CHUNK)], dist_v[b2],
                              isems[b2]).wait()
        pltpu.make_async_copy(mol_hbm.at[pl.ds(arow, ROWS_PER_CHUNK), :],
                              first_v[b4], isems[b2]).wait()
        pltpu.sync_copy(acc_sh.at[pl.ds(abase, CHUNK)], v_v[b4])

    def drain_bins(b4):
        for j in range(ROWS_PER_CHUNK):
            pltpu.make_async_copy(v_v[b4].at[pl.ds(j * 128, 128)],
                                  bins_sh.at[first_v[b4].at[j]],
                                  ssems[b4]).wait()

    bin_start(0, 0, 0)
    for m in range(MSUB):
        b2, b4 = m % 2, m % 4
        bin_wait(m, b2, b4)
        if m >= 3:
            drain_bins((m + 1) % 4)
        if m + 1 < MSUB:
            bin_start(m + 1, (m + 1) % 2, (m + 1) % 4)
        va, vq = v_v[b4], dist_v[b2]

        def binner(i, c):
            s = pl.ds(i * L, L)
            va[s] = (0.5 * va[s]) * vq[s]
            return c

        lax.fori_loop(0, CHUNK // L, binner, 0, unroll=4)
        for j in range(ROWS_PER_CHUNK):
            pltpu.async_copy(v_v[b4].at[pl.ds(j * 128, 128)],
                             bins_sh.at[first_v[b4].at[j]], ssems[b4],
                             add=True)

    drain_bins(1)
    drain_bins(2)
    drain_bins(3)
    acc_dump.wait()
    plsc.subcore_barrier()

    @pl.when(sid == 0)
    def _():
        pltpu.sync_copy(bins_sh, bins_hbm.at[pl.ds(cid * NM_PAD, NM_PAD)])


@functools.partial(
    pl.kernel,
    mesh=_mesh,
    out_type=jax.ShapeDtypeStruct((NA_PAD,), jnp.float32),
    compiler_params=pltpu.CompilerParams(needs_layout_passes=False),
    scratch_types=[
        pltpu.VMEM((APW,), jnp.float32),
        pltpu.VMEM((APW,), jnp.float32),
        pltpu.VMEM((APW,), jnp.float32),
    ],
)
def _volt_kernel(part_hbm, volt_hbm, p0_v, p1_v, v_v):
    cid = lax.axis_index("c")
    sid = lax.axis_index("s")
    wid = cid * NS + sid
    base = wid * APW

    pltpu.sync_copy(part_hbm.at[pl.ds(base, APW)], p0_v)
    pltpu.sync_copy(part_hbm.at[pl.ds(NA_PAD + base, APW)], p1_v)

    def inner(i, c):
        s = pl.ds(i * L, L)
        v_v[s] = p0_v[s] + p1_v[s]
        return c

    lax.fori_loop(0, APW // L, inner, 0, unroll=4)

    pltpu.sync_copy(v_v, volt_hbm.at[pl.ds(base, APW)])


def kernel(charges, pair_dist, pair_first, pair_second, mol_index, n_molecules):
    q = charges.reshape(NA)
    first_2d = pair_first.reshape(NP // 128, 128)
    pada = NA_PAD - NA
    q_pad = jnp.concatenate([q, jnp.zeros((pada,), jnp.float32)])
    mol_p = jnp.concatenate(
        [mol_index, jnp.full((pada,), NM_PAD - 1, jnp.int32)]).reshape(NA_PAD // 128, 128)
    zeros_acc = jnp.zeros((NA_PAD,), jnp.float32)

    part, bins = _pairs_kernel(q_pad, pair_dist, first_2d, pair_second,
                               mol_p, zeros_acc)
    volt = _volt_kernel(part)

    voltage_atom = volt[:NA].reshape(NA, 1)
    coulomb_molecules = (bins[:NM] + bins[NM_PAD:NM_PAD + NM]).reshape(NM, 1)
    return (coulomb_molecules, voltage_atom)


# speedup vs baseline: 1.0098x; 1.0074x over previous
import functools

import jax
import jax.numpy as jnp
from jax import lax
from jax.experimental import pallas as pl
from jax.experimental.pallas import tpu as pltpu
from jax.experimental.pallas import tpu_sc as plsc

F = 14.399645

NA = 100000
NP = 6400000
NM = 1000

NC, NS, L = 2, 16, 16
NW = NC * NS

NA_PAD = 131072
CHUNK = 1024
ROWS_PER_CHUNK = CHUNK // 128
NCH_TOTAL = NP // CHUNK
NCH_MAIN = 195
NCH_PIPE = 192
PPW = NCH_MAIN * CHUNK
NCH_EXTRA = NCH_TOTAL - NW * NCH_MAIN

NM_PAD = 1024
APW = NA_PAD // NW
BROWS = APW // 128

_mesh = plsc.VectorSubcoreMesh(core_axis_name="c", subcore_axis_name="s")


@functools.partial(
    pl.kernel,
    mesh=_mesh,
    out_type=(
        jax.ShapeDtypeStruct((NC * NA_PAD,), jnp.float32),
        jax.ShapeDtypeStruct((NC * NM_PAD,), jnp.float32),
        jax.ShapeDtypeStruct((NA_PAD,), jnp.float32),
        jax.ShapeDtypeStruct((NC * L,), jnp.int32),
    ),
    compiler_params=pltpu.CompilerParams(needs_layout_passes=False),
    scratch_types=[
        pltpu.VMEM((NA,), jnp.float32),
        pltpu.VMEM((CHUNK,), jnp.int32),
        pltpu.VMEM((CHUNK,), jnp.int32),
        pltpu.VMEM((CHUNK,), jnp.float32),
        pltpu.VMEM((CHUNK,), jnp.float32),
        pltpu.VMEM((ROWS_PER_CHUNK, 128), jnp.int32),
        pltpu.VMEM((ROWS_PER_CHUNK, 128), jnp.int32),
        pltpu.VMEM((ROWS_PER_CHUNK, 128), jnp.int32),
        pltpu.VMEM((ROWS_PER_CHUNK, 128), jnp.int32),
        pltpu.VMEM((CHUNK,), jnp.float32),
        pltpu.VMEM((CHUNK,), jnp.float32),
        pltpu.VMEM((CHUNK,), jnp.float32),
        pltpu.VMEM((CHUNK,), jnp.float32),
        pltpu.VMEM_SHARED((NA_PAD,), jnp.float32),
        pltpu.VMEM_SHARED((NM_PAD,), jnp.float32),
        pltpu.SemaphoreType.DMA,
        pltpu.SemaphoreType.DMA,
        pltpu.SemaphoreType.DMA,
        pltpu.SemaphoreType.DMA,
        pltpu.SemaphoreType.DMA,
        pltpu.SemaphoreType.DMA,
        pltpu.SemaphoreType.DMA,
    ],
)
def _pairs_kernel(q_hbm, dist_hbm, first_hbm, sec_hbm, mol_hbm, zeros_hbm,
                  part_hbm, bins_hbm, volt_hbm, flags_hbm,
                  ch_v, sec0_v, sec1_v, dist0_v, dist1_v,
                  first0_v, first1_v, first2_v, first3_v,
                  v0_v, v1_v, v2_v, v3_v, acc_sh, bins_sh,
                  isem0, isem1, ssem0, ssem1, ssem2, ssem3, dsem):
    cid = lax.axis_index("c")
    sid = lax.axis_index("s")
    wid = cid * NS + sid
    sec_v = (sec0_v, sec1_v)
    dist_v = (dist0_v, dist1_v)
    first_v = (first0_v, first1_v, first2_v, first3_v)
    v_v = (v0_v, v1_v, v2_v, v3_v)
    isems = (isem0, isem1)
    ssems = (ssem0, ssem1, ssem2, ssem3)

    pltpu.sync_copy(q_hbm.at[pl.ds(0, NA)], ch_v)

    @pl.when(sid == 0)
    def _():
        pltpu.sync_copy(zeros_hbm, acc_sh)
        pltpu.sync_copy(zeros_hbm.at[pl.ds(0, NM_PAD)], bins_sh)
        sec0_v[pl.ds(0, L)] = jnp.zeros((L,), jnp.int32)
        pltpu.sync_copy(sec0_v.at[pl.ds(0, L)],
                        flags_hbm.at[pl.ds(cid * L, L)])

    plsc.subcore_barrier()

    def chunk_off(k):
        return wid * PPW + k * CHUNK

    def chunk_row(k):
        return wid * (PPW // 128) + k * ROWS_PER_CHUNK

    def extra_off():
        return (NW * NCH_MAIN + wid) * CHUNK

    def extra_row():
        return NW * NCH_MAIN * ROWS_PER_CHUNK + wid * ROWS_PER_CHUNK

    def start_inputs(off, row, b2, b4):
        pltpu.async_copy(sec_hbm.at[pl.ds(off, CHUNK)], sec_v[b2], isems[b2])
        pltpu.async_copy(dist_hbm.at[pl.ds(off, CHUNK)], dist_v[b2], isems[b2])
        pltpu.async_copy(first_hbm.at[pl.ds(row, ROWS_PER_CHUNK), :],
                         first_v[b4], isems[b2])

    def wait_inputs(off, row, b2, b4):
        pltpu.make_async_copy(sec_hbm.at[pl.ds(off, CHUNK)], sec_v[b2],
                              isems[b2]).wait()
        pltpu.make_async_copy(dist_hbm.at[pl.ds(off, CHUNK)], dist_v[b2],
                              isems[b2]).wait()
        pltpu.make_async_copy(first_hbm.at[pl.ds(row, ROWS_PER_CHUNK), :],
                              first_v[b4], isems[b2]).wait()

    def drain_scatter(b4):
        for j in range(ROWS_PER_CHUNK):
            pltpu.make_async_copy(v_v[b4].at[pl.ds(j * 128, 128)],
                                  acc_sh.at[first_v[b4].at[j]],
                                  ssems[b4]).wait()

    def compute(b2, b4):
        sv, dv, vv = sec_v[b2], dist_v[b2], v_v[b4]

        def inner(i, c):
            s = pl.ds(i * L, L)
            idx = sv[s]
            q = plsc.load_gather(ch_v, [idx])
            d = dv[s]
            vv[s] = (F * q) / d
            return c

        lax.fori_loop(0, CHUNK // L, inner, 0, unroll=4)

    def issue_scatter(b4):
        for j in range(ROWS_PER_CHUNK):
            pltpu.async_copy(v_v[b4].at[pl.ds(j * 128, 128)],
                             acc_sh.at[first_v[b4].at[j]], ssems[b4],
                             add=True)

    start_inputs(chunk_off(0), chunk_row(0), 0, 0)

    def pair_body(t, carry):
        for u in range(4):
            k = 4 * t + u
            b2, b4 = u % 2, u
            wait_inputs(chunk_off(k), chunk_row(k), b2, b4)

            @pl.when(k >= 3)
            def _():
                drain_scatter((u + 1) % 4)

            @pl.when(k + 1 < NCH_PIPE)
            def _():
                start_inputs(chunk_off(k + 1), chunk_row(k + 1),
                             (u + 1) % 2, (u + 1) % 4)

            compute(b2, b4)
            issue_scatter(b4)
        return carry

    lax.fori_loop(0, NCH_PIPE // 4, pair_body, 0)
    drain_scatter(1)
    drain_scatter(2)
    drain_scatter(3)

    for j, k in enumerate(range(NCH_PIPE, NCH_MAIN)):
        pltpu.sync_copy(sec_hbm.at[pl.ds(chunk_off(k), CHUNK)], sec_v[j % 2])
        pltpu.sync_copy(dist_hbm.at[pl.ds(chunk_off(k), CHUNK)], dist_v[j % 2])
        pltpu.sync_copy(first_hbm.at[pl.ds(chunk_row(k), ROWS_PER_CHUNK), :],
                        first_v[j])
        compute(j % 2, j)
        issue_scatter(j)

    @pl.when(wid < NCH_EXTRA)
    def _():
        pltpu.sync_copy(sec_hbm.at[pl.ds(extra_off(), CHUNK)], sec_v[0])
        pltpu.sync_copy(dist_hbm.at[pl.ds(extra_off(), CHUNK)], dist_v[0])
        pltpu.sync_copy(first_hbm.at[pl.ds(extra_row(), ROWS_PER_CHUNK), :],
                        first_v[3])
        compute(0, 3)
        issue_scatter(3)

    drain_scatter(0)
    drain_scatter(1)
    drain_scatter(2)

    @pl.when(wid < NCH_EXTRA)
    def _():
        drain_scatter(3)

    plsc.subcore_barrier()

    seg = NA_PAD // NS
    acc_dump = pltpu.async_copy(
        acc_sh.at[pl.ds(sid * seg, seg)],
        part_hbm.at[pl.ds(cid * NA_PAD + sid * seg, seg)], dsem)

    MSUB = seg // CHUNK

    def bin_start(m, b2, b4):
        abase = sid * seg + m * CHUNK
        arow = sid * (seg // 128) + m * ROWS_PER_CHUNK
        pltpu.async_copy(q_hbm.at[pl.ds(abase, CHUNK)], dist_v[b2], isems[b2])
        pltpu.async_copy(mol_hbm.at[pl.ds(arow, ROWS_PER_CHUNK), :],
                         first_v[b4], isems[b2])

    def bin_wait(m, b2, b4):
        abase = sid * seg + m * CHUNK
        arow = sid * (seg // 128) + m * ROWS_PER_CHUNK
        pltpu.make_async_copy(q_hbm.at[pl.ds(abase, CHUNK)], dist_v[b2],
                              isems[b2]).wait()
        pltpu.make_async_copy(mol_hbm.at[pl.ds(arow, ROWS_PER_CHUNK), :],
                              first_v[b4], isems[b2]).wait()
        pltpu.sync_copy(acc_sh.at[pl.ds(abase, CHUNK)], v_v[b4])

    def drain_bins(b4):
        for j in range(ROWS_PER_CHUNK):
            pltpu.make_async_copy(v_v[b4].at[pl.ds(j * 128, 128)],
                                  bins_sh.at[first_v[b4].at[j]],
                                  ssems[b4]).wait()

    bin_start(0, 0, 0)
    for m in range(MSUB):
        b2, b4 = m % 2, m % 4
        bin_wait(m, b2, b4)
        if m >= 3:
            drain_bins((m + 1) % 4)
        if m + 1 < MSUB:
            bin_start(m + 1, (m + 1) % 2, (m + 1) % 4)
        va, vq = v_v[b4], dist_v[b2]

        def binner(i, c):
            s = pl.ds(i * L, L)
            va[s] = (0.5 * va[s]) * vq[s]
            return c

        lax.fori_loop(0, CHUNK // L, binner, 0, unroll=4)
        for j in range(ROWS_PER_CHUNK):
            pltpu.async_copy(v_v[b4].at[pl.ds(j * 128, 128)],
                             bins_sh.at[first_v[b4].at[j]], ssems[b4],
                             add=True)

    drain_bins(1)
    drain_bins(2)
    drain_bins(3)
    acc_dump.wait()
    plsc.subcore_barrier()

    @pl.when(sid == 0)
    def _():
        pltpu.sync_copy(bins_sh, bins_hbm.at[pl.ds(cid * NM_PAD, NM_PAD)])
        sec0_v[pl.ds(0, L)] = jnp.ones((L,), jnp.int32)
        pltpu.sync_copy(sec0_v.at[pl.ds(0, L)],
                        flags_hbm.at[pl.ds(cid * L, L)])

    def poll_cond(carry):
        f, it = carry
        return jnp.logical_and(f != 1, it < jnp.int32(500000))

    def poll_body(carry):
        f, it = carry
        pltpu.sync_copy(flags_hbm.at[pl.ds((1 - cid) * L, L)],
                        sec1_v.at[pl.ds(0, L)])
        vf = sec1_v[pl.ds(0, L)]
        return (jnp.max(vf), it + jnp.int32(1))

    lax.while_loop(poll_cond, poll_body, (jnp.int32(0), jnp.int32(0)))

    half = NA_PAD // NC
    vseg = half // NS
    VSUB = vseg // CHUNK
    vbase = cid * half + sid * vseg

    def volt_in_start(s2, b):
        off = vbase + s2 * CHUNK
        pltpu.async_copy(
            part_hbm.at[pl.ds((1 - cid) * NA_PAD + off, CHUNK)],
            dist_v[b], isems[b])

    def volt_in_wait(s2, b):
        off = vbase + s2 * CHUNK
        pltpu.make_async_copy(
            part_hbm.at[pl.ds((1 - cid) * NA_PAD + off, CHUNK)],
            dist_v[b], isems[b]).wait()

    def volt_out_wait(s2, b):
        off = vbase + s2 * CHUNK
        pltpu.make_async_copy(v_v[b], volt_hbm.at[pl.ds(off, CHUNK)],
                              ssems[b]).wait()

    volt_in_start(0, 0)
    for s2 in range(VSUB):
        b = s2 % 2
        volt_in_wait(s2, b)
        if s2 + 1 < VSUB:
            volt_in_start(s2 + 1, 1 - b)
        if s2 >= 2:
            volt_out_wait(s2 - 2, b)
        off = vbase + s2 * CHUNK
        pltpu.sync_copy(acc_sh.at[pl.ds(off, CHUNK)], v_v[b])
        va, vd = v_v[b], dist_v[b]

        def vadd(i, c):
            s = pl.ds(i * L, L)
            va[s] = va[s] + vd[s]
            return c

        lax.fori_loop(0, CHUNK // L, vadd, 0, unroll=4)
        pltpu.async_copy(v_v[b], volt_hbm.at[pl.ds(off, CHUNK)], ssems[b])

    volt_out_wait(VSUB - 2, 0)
    volt_out_wait(VSUB - 1, 1)


def kernel(charges, pair_dist, pair_first, pair_second, mol_index, n_molecules):
    q = charges.reshape(NA)
    first_2d = pair_first.reshape(NP // 128, 128)
    pada = NA_PAD - NA
    q_pad = jnp.concatenate([q, jnp.zeros((pada,), jnp.float32)])
    mol_p = jnp.concatenate(
        [mol_index, jnp.full((pada,), NM_PAD - 1, jnp.int32)]).reshape(NA_PAD // 128, 128)
    zeros_acc = jnp.zeros((NA_PAD,), jnp.float32)

    part, bins, volt, flags = _pairs_kernel(q_pad, pair_dist, first_2d,
                                            pair_second, mol_p, zeros_acc)
    del part, flags

    voltage_atom = volt[:NA].reshape(NA, 1)
    coulomb_molecules = (bins[:NM] + bins[NM_PAD:NM_PAD + NM]).reshape(NM, 1)
    return (coulomb_molecules, voltage_atom)
